# baseline (speedup 1.0000x reference)
import functools
import math

import jax
import jax.numpy as jnp
import numpy as np
from jax import lax
from jax.experimental import pallas as pl
from jax.experimental.pallas import tpu as pltpu
from jax.experimental.pallas import tpu_sc as plsc

VOCAB = 100000
D_MODEL = 768
MAX_LEN = 4096
BATCH = 4
SCALE = math.sqrt(D_MODEL)

N_ROWS = MAX_LEN * BATCH
NW = 32
ROWS_PER_W = N_ROWS // NW
CHUNK_ROWS = 64
N_CHUNKS = ROWS_PER_W // CHUNK_ROWS
L_PER_CHUNK = CHUNK_ROWS // BATCH
LANES = 16
C_PER_ROW = D_MODEL // LANES


def _pe_table():
    pe = np.zeros((MAX_LEN, D_MODEL), dtype=np.float32)
    position = np.arange(0, MAX_LEN, dtype=np.float32)[:, None]
    div_term = np.exp(
        np.arange(0, D_MODEL, 2, dtype=np.float32) * (-math.log(10000.0) / D_MODEL)
    )
    pe[:, 0::2] = np.sin(position * div_term)
    pe[:, 1::2] = np.cos(position * div_term)
    return jnp.asarray(pe)


_MESH = plsc.VectorSubcoreMesh(core_axis_name="c", subcore_axis_name="s")


@functools.partial(
    pl.kernel,
    mesh=_MESH,
    out_type=jax.ShapeDtypeStruct((N_ROWS, D_MODEL), jnp.float32),
    scratch_types=[
        pltpu.VMEM((N_CHUNKS, CHUNK_ROWS), jnp.int32),
        pltpu.VMEM((CHUNK_ROWS, D_MODEL), jnp.float32),
        pltpu.VMEM((L_PER_CHUNK, D_MODEL), jnp.float32),
        pltpu.SemaphoreType.DMA,
    ],
)
def _embed_sc(table_hbm, idx_hbm, pe_hbm, out_hbm, idx_v, rows_v, pe_v, sem):
    wid = lax.axis_index("s") * 2 + lax.axis_index("c")
    base_row = wid * ROWS_PER_W
    base_l = wid * (ROWS_PER_W // BATCH)

    pltpu.sync_copy(idx_hbm.at[pl.ds(wid * N_CHUNKS, N_CHUNKS)], idx_v)

    def chunk_body(g, carry):
        l0 = base_l + g * L_PER_CHUNK
        r0 = base_row + g * CHUNK_ROWS
        pltpu.sync_copy(pe_hbm.at[pl.ds(l0, L_PER_CHUNK)], pe_v)
        pltpu.async_copy(table_hbm.at[idx_v.at[g]], rows_v, sem).wait()

        def l_body(li, c2):
            for c in range(C_PER_ROW):
                pe_c = pe_v[li, pl.ds(c * LANES, LANES)]
                for b in range(BATCH):
                    r = li * BATCH + b
                    rows_v[r, pl.ds(c * LANES, LANES)] = (
                        rows_v[r, pl.ds(c * LANES, LANES)] * SCALE + pe_c
                    )
            return c2

        lax.fori_loop(0, L_PER_CHUNK, l_body, 0)
        pltpu.sync_copy(rows_v, out_hbm.at[pl.ds(r0, CHUNK_ROWS)])
        return carry

    lax.fori_loop(0, N_CHUNKS, chunk_body, 0)


def kernel(encoded_words, embed_weight):
    idx2d = encoded_words.reshape(NW * N_CHUNKS, CHUNK_ROWS)
    pe = _pe_table()
    out = _embed_sc(embed_weight, idx2d, pe)
    return out.reshape(MAX_LEN, BATCH, D_MODEL)
